# initial kernel scaffold (unmeasured)
import jax
import jax.numpy as jnp
from jax import lax
from jax.experimental import pallas as pl
from jax.experimental.pallas import tpu as pltpu


def kernel(
    x,
):
    def body(*refs):
        pass

    out_shape = jax.ShapeDtypeStruct(..., jnp.float32)
    return pl.pallas_call(body, out_shape=out_shape)(...)


# baseline (device time: 18008 ns/iter reference)
import jax
import jax.numpy as jnp
from jax import lax
from jax.experimental import pallas as pl
from jax.experimental.pallas import tpu as pltpu

N_DEV = 8


def _local_cumprod(y, m, n):
    d = 1
    while d < m:
        top = jnp.ones((d, n), jnp.float32)
        y = y * jnp.concatenate([top, y[: m - d, :]], axis=0)
        d *= 2
    return y


def kernel(x):
    m, n = x.shape

    def body(x_ref, out_ref, send_buf, recv_buf, send_sems, recv_sems):
        my = lax.axis_index("i")

        barrier_sem = pltpu.get_barrier_semaphore()
        for r in range(N_DEV):
            @pl.when(my != r)
            def _():
                pl.semaphore_signal(
                    barrier_sem, inc=1,
                    device_id=(r,), device_id_type=pl.DeviceIdType.MESH,
                )
        pl.semaphore_wait(barrier_sem, N_DEV - 1)

        out_ref[:, :] = _local_cumprod(x_ref[:, :], m, n)

        send_buf[:, :] = out_ref[m - 1 : m, :]

        for d in range(1, N_DEV):
            @pl.when(my + d < N_DEV)
            def _():
                rdma = pltpu.make_async_remote_copy(
                    src_ref=send_buf,
                    dst_ref=recv_buf.at[d - 1],
                    send_sem=send_sems.at[d - 1],
                    recv_sem=recv_sems.at[d - 1],
                    device_id=(my + d,),
                    device_id_type=pl.DeviceIdType.MESH,
                )
                rdma.start()

        for d in range(1, N_DEV):
            @pl.when(d <= my)
            def _():
                rdma = pltpu.make_async_remote_copy(
                    src_ref=send_buf,
                    dst_ref=recv_buf.at[d - 1],
                    send_sem=send_sems.at[d - 1],
                    recv_sem=recv_sems.at[d - 1],
                    device_id=(my - d,),
                    device_id_type=pl.DeviceIdType.MESH,
                )
                rdma.wait_recv()

        prefix = jnp.ones((1, n), jnp.float32)
        for d in range(1, N_DEV):
            prefix = prefix * jnp.where(d <= my, recv_buf[d - 1], 1.0)

        out_ref[:, :] = out_ref[:, :] * prefix

        for d in range(1, N_DEV):
            @pl.when(my + d < N_DEV)
            def _():
                rdma = pltpu.make_async_remote_copy(
                    src_ref=send_buf,
                    dst_ref=recv_buf.at[d - 1],
                    send_sem=send_sems.at[d - 1],
                    recv_sem=recv_sems.at[d - 1],
                    device_id=(my + d,),
                    device_id_type=pl.DeviceIdType.MESH,
                )
                rdma.wait_send()

    return pl.pallas_call(
        body,
        out_shape=jax.ShapeDtypeStruct((m, n), jnp.float32),
        in_specs=[pl.BlockSpec(memory_space=pltpu.VMEM)],
        out_specs=pl.BlockSpec(memory_space=pltpu.VMEM),
        scratch_shapes=[
            pltpu.VMEM((1, n), jnp.float32),
            pltpu.VMEM((N_DEV - 1, 1, n), jnp.float32),
            pltpu.SemaphoreType.DMA((N_DEV - 1,)),
            pltpu.SemaphoreType.DMA((N_DEV - 1,)),
        ],
        compiler_params=pltpu.CompilerParams(collective_id=0),
    )(x)
